# baseline (speedup 1.0000x reference)
import functools

import jax
import jax.numpy as jnp
from jax import lax
from jax.experimental import pallas as pl
from jax.experimental.pallas import tpu as pltpu
from jax.experimental.pallas import tpu_sc as plsc

N = 10000
E = 160000
D_IN = 256
D_H = 512
D_OUT = 40
DC = 128
DC2 = 128
NC = 2
NS = 16
RPT = N // NS
B = 128
EP = 1280
EB = EP // NS
HEB = EB // 2
N16 = N + 16

_mesh = functools.partial(
    plsc.VectorSubcoreMesh, core_axis_name="c", subcore_axis_name="s")


def _fill_const(buf, rows, cols, val):
  nz = cols // 16

  def body(r, _):
    for j in range(nz):
      buf[r, j * 16:(j + 1) * 16] = jnp.full((16,), val, jnp.float32)
    return 0

  lax.fori_loop(0, rows, body, 0)


def _zero_acc(zbuf, acc, r0):
  for j in range(25):
    pltpu.sync_copy(zbuf, acc.at[pl.ds(r0 + j * 25, 25)])


def _pipelined_pass(tab, acc, sall, dall, m0, m1, g0, g1, s0, s1, nb):
  def gfire(t, m, g):
    pltpu.async_copy(tab.at[sall.at[t]], m, g)

  def gwait(t, m, g):
    pltpu.make_async_copy(tab.at[sall.at[t]], m, g).wait()

  def sfire(t, m, s):
    pltpu.async_copy(m, acc.at[dall.at[t]], s, add=True)

  def swait(t, m, s):
    pltpu.make_async_copy(m, acc.at[dall.at[t]], s).wait()

  def body(t, _):
    gfire(t, m0, g0)
    gwait(t, m0, g0)
    sfire(t, m0, s0)
    swait(t, m0, s0)
    return 0

  lax.fori_loop(0, nb, body, 0)
  del m1, g1, s1


def _hist_call(src2d, dst2d):
  @functools.partial(
      pl.kernel,
      out_type=jax.ShapeDtypeStruct((2, NS, RPT, DC), jnp.float32),
      mesh=_mesh(),
      scratch_types=[
          pltpu.VMEM((HEB, B), jnp.int32),
          pltpu.VMEM((B, DC), jnp.float32),
          pltpu.VMEM((25, DC), jnp.float32),
          pltpu.VMEM_SHARED((N16, DC), jnp.float32),
          pltpu.SemaphoreType.DMA,
          pltpu.SemaphoreType.DMA,
      ],
  )
  def k(src_hbm, dst_hbm, out_hbm, dall, onesb, zbuf, acc, s0, s1):
    cid = lax.axis_index("c")
    sid = lax.axis_index("s")
    _fill_const(onesb, B, DC, 1.0)
    _fill_const(zbuf, 25, DC, 0.0)
    r0 = sid * RPT
    _zero_acc(zbuf, acc, r0)
    row0 = sid * EB
    plsc.subcore_barrier()

    def sfire(t, s):
      pltpu.async_copy(onesb, acc.at[dall.at[t]], s, add=True)

    def swait(t, s):
      pltpu.make_async_copy(onesb, acc.at[dall.at[t]], s).wait()

    def body(i, _):
      t0 = 2 * i
      t1 = t0 + 1
      swait(t0, s0)

      @pl.when(t0 + 2 < HEB)
      def _():
        sfire(t0 + 2, s0)

      swait(t1, s1)

      @pl.when(t1 + 2 < HEB)
      def _():
        sfire(t1 + 2, s1)

      return 0

    for h in range(2):
      @pl.when(cid == 0)
      def _():
        pltpu.sync_copy(src_hbm.at[pl.ds(row0 + h * HEB, HEB)], dall)

      @pl.when(cid == 1)
      def _():
        pltpu.sync_copy(dst_hbm.at[pl.ds(row0 + h * HEB, HEB)], dall)

      sfire(0, s0)
      sfire(1, s1)
      lax.fori_loop(0, HEB // 2, body, 0)

    plsc.subcore_barrier()
    pltpu.sync_copy(acc.at[pl.ds(r0, RPT)], out_hbm.at[cid, sid])

  return k(src2d, dst2d)


def _agg_cols_call(src2d, dst2d, tables):
  nt = len(tables)
  cpc = nt // 2

  @functools.partial(
      pl.kernel,
      out_type=[jax.ShapeDtypeStruct((NS, RPT, DC), jnp.float32)] * nt,
      mesh=_mesh(),
      scratch_types=[
          pltpu.VMEM((HEB, B), jnp.int32),
          pltpu.VMEM((HEB, B), jnp.int32),
          pltpu.VMEM((B, DC), jnp.float32),
          pltpu.VMEM((B, DC), jnp.float32),
          pltpu.VMEM((25, DC), jnp.float32),
          pltpu.VMEM_SHARED((N16, DC), jnp.float32),
          pltpu.SemaphoreType.DMA,
          pltpu.SemaphoreType.DMA,
          pltpu.SemaphoreType.DMA,
          pltpu.SemaphoreType.DMA,
      ],
  )
  def k(src_hbm, dst_hbm, *rest):
    tabs = rest[:nt]
    outs = rest[nt:2 * nt]
    (sall, dall, m0, m1, zbuf, acc, g0, g1, s0, s1) = rest[2 * nt:]
    cid = lax.axis_index("c")
    sid = lax.axis_index("s")
    _fill_const(zbuf, 25, DC, 0.0)
    r0 = sid * RPT
    row0 = sid * EB

    def copy_out(out):
      pltpu.sync_copy(acc.at[pl.ds(r0, RPT)], out.at[sid])

    def do_chunk(tab):
      for h in range(2):
        pltpu.sync_copy(src_hbm.at[pl.ds(row0 + h * HEB, HEB)], sall)
        pltpu.sync_copy(dst_hbm.at[pl.ds(row0 + h * HEB, HEB)], dall)
        _pipelined_pass(tab, acc, sall, dall, m0, m1, g0, g1, s0, s1, HEB)

    _zero_acc(zbuf, acc, r0)
    plsc.subcore_barrier()
    for i in range(cpc):
      @pl.when(cid == 0)
      def _():
        do_chunk(tabs[i])

      @pl.when(cid == 1)
      def _():
        do_chunk(tabs[cpc + i])

      plsc.subcore_barrier()

      @pl.when(cid == 0)
      def _():
        copy_out(outs[i])

      @pl.when(cid == 1)
      def _():
        copy_out(outs[cpc + i])

      if i < cpc - 1:
        _zero_acc(zbuf, acc, r0)
        plsc.subcore_barrier()

  return k(src2d, dst2d, *tables)


def _agg_edges_call(src2d, dst2d, table):
  eb = EP // (NC * NS)

  @functools.partial(
      pl.kernel,
      out_type=jax.ShapeDtypeStruct((2, NS, RPT, DC2), jnp.float32),
      mesh=_mesh(),
      scratch_types=[
          pltpu.VMEM((eb, B), jnp.int32),
          pltpu.VMEM((eb, B), jnp.int32),
          pltpu.VMEM((B, DC2), jnp.float32),
          pltpu.VMEM((B, DC2), jnp.float32),
          pltpu.VMEM((25, DC2), jnp.float32),
          pltpu.VMEM_SHARED((N16, DC2), jnp.float32),
          pltpu.SemaphoreType.DMA,
          pltpu.SemaphoreType.DMA,
          pltpu.SemaphoreType.DMA,
          pltpu.SemaphoreType.DMA,
      ],
  )
  def k(src_hbm, dst_hbm, tab, out_hbm, sall, dall, m0, m1, zbuf, acc, g0,
        g1, s0, s1):
    cid = lax.axis_index("c")
    sid = lax.axis_index("s")
    _fill_const(zbuf, 25, DC2, 0.0)
    r0 = sid * RPT
    row0 = (cid * NS + sid) * eb
    pltpu.sync_copy(src_hbm.at[pl.ds(row0, eb)], sall)
    pltpu.sync_copy(dst_hbm.at[pl.ds(row0, eb)], dall)
    _zero_acc(zbuf, acc, r0)
    plsc.subcore_barrier()
    _pipelined_pass(tab, acc, sall, dall, m0, m1, g0, g1, s0, s1, eb)
    plsc.subcore_barrier()
    pltpu.sync_copy(acc.at[pl.ds(r0, RPT)], out_hbm.at[cid, sid])

  return k(src2d, dst2d, table)


_R = 1000


def _norms(hist_blk):
  deg_o = hist_blk[0, :, 0:1]
  deg_i = hist_blk[1, :, 0:1]
  return (lax.rsqrt(jnp.maximum(deg_o, 1.0)),
          lax.rsqrt(jnp.maximum(deg_i, 1.0)))


def _ep0_call(x, hist):
  def body(x_ref, h_ref, o0_ref, o1_ref):
    no, _ = _norms(h_ref[...])
    xs = x_ref[...] * no
    o0_ref[...] = xs[:, :DC]
    o1_ref[...] = xs[:, DC:]

  return pl.pallas_call(
      body,
      grid=(N // _R,),
      in_specs=[
          pl.BlockSpec((_R, D_IN), lambda i: (i, 0)),
          pl.BlockSpec((2, _R, DC), lambda i: (0, i, 0)),
      ],
      out_specs=[pl.BlockSpec((_R, DC), lambda i: (i, 0))] * 2,
      out_shape=[jax.ShapeDtypeStruct((N16, DC), jnp.float32)] * 2,
  )(x, hist)


def _mm0_call(a0, a1, hist, w0, b0):
  def body(a0_ref, a1_ref, h_ref, w_ref, b_ref, *o_refs):
    no, ni = _norms(h_ref[...])
    a = jnp.concatenate([a0_ref[...], a1_ref[...]], axis=1) * ni
    h = jnp.dot(a, w_ref[...], preferred_element_type=jnp.float32,
                precision=lax.Precision.HIGHEST)
    h = jnp.maximum(h + b_ref[...], 0.0) * no
    for j in range(4):
      o_refs[j][...] = h[:, j * DC:(j + 1) * DC]

  return pl.pallas_call(
      body,
      grid=(N // _R,),
      in_specs=[
          pl.BlockSpec((_R, DC), lambda i: (i, 0)),
          pl.BlockSpec((_R, DC), lambda i: (i, 0)),
          pl.BlockSpec((2, _R, DC), lambda i: (0, i, 0)),
          pl.BlockSpec((D_IN, D_H), lambda i: (0, 0)),
          pl.BlockSpec((1, D_H), lambda i: (0, 0)),
      ],
      out_specs=[pl.BlockSpec((_R, DC), lambda i: (i, 0))] * 4,
      out_shape=[jax.ShapeDtypeStruct((N16, DC), jnp.float32)] * 4,
  )(a0, a1, hist, w0, b0)


def _mm12_call(aggs, hist, w1, b1, w2p):
  def body(a0_ref, a1_ref, a2_ref, a3_ref, h_ref, w1_ref, b1_ref, w2_ref,
           o_ref):
    no, ni = _norms(h_ref[...])
    a = jnp.concatenate(
        [a0_ref[...], a1_ref[...], a2_ref[...], a3_ref[...]], axis=1) * ni
    t = jnp.dot(a, w1_ref[...], preferred_element_type=jnp.float32,
                precision=lax.Precision.HIGHEST)
    t = jnp.maximum(t + b1_ref[...], 0.0) * no
    o_ref[...] = jnp.dot(t, w2_ref[...], preferred_element_type=jnp.float32,
                         precision=lax.Precision.HIGHEST)

  return pl.pallas_call(
      body,
      grid=(N // _R,),
      in_specs=[pl.BlockSpec((_R, DC), lambda i: (i, 0))] * 4 + [
          pl.BlockSpec((2, _R, DC), lambda i: (0, i, 0)),
          pl.BlockSpec((D_H, D_H), lambda i: (0, 0)),
          pl.BlockSpec((1, D_H), lambda i: (0, 0)),
          pl.BlockSpec((D_H, DC2), lambda i: (0, 0)),
      ],
      out_specs=pl.BlockSpec((_R, DC2), lambda i: (i, 0)),
      out_shape=jax.ShapeDtypeStruct((N16, DC2), jnp.float32),
  )(*aggs, hist, w1, b1, w2p)


def _final_call(p, hist, b2):
  def body(p_ref, h_ref, b_ref, o_ref):
    _, ni = _norms(h_ref[...])
    s = (p_ref[0] + p_ref[1])[:, :D_OUT]
    o_ref[...] = s * ni + b_ref[...]

  return pl.pallas_call(
      body,
      grid=(N // _R,),
      in_specs=[
          pl.BlockSpec((2, _R, DC2), lambda i: (0, i, 0)),
          pl.BlockSpec((2, _R, DC), lambda i: (0, i, 0)),
          pl.BlockSpec((1, D_OUT), lambda i: (0, 0)),
      ],
      out_specs=pl.BlockSpec((_R, D_OUT), lambda i: (i, 0)),
      out_shape=jax.ShapeDtypeStruct((N, D_OUT), jnp.float32),
  )(p, hist, b2)


def kernel(features, edge_index, W0, b0, W1, b1, W2, b2):
  ei = edge_index.astype(jnp.int32)
  pad = jnp.full((EP * B - E,), N, jnp.int32)
  src2d = jnp.concatenate([ei[0], pad]).reshape(EP, B)
  dst2d = jnp.concatenate([ei[1], pad]).reshape(EP, B)
  hist = _hist_call(src2d, dst2d).reshape(2, N, DC)

  xs0, xs1 = _ep0_call(features, hist)
  a00, a01 = _agg_cols_call(src2d, dst2d, [xs0, xs1])
  h1 = _mm0_call(a00.reshape(N, DC), a01.reshape(N, DC), hist, W0,
                 b0.reshape(1, D_H))

  a1 = _agg_cols_call(src2d, dst2d, list(h1))
  a1 = [a.reshape(N, DC) for a in a1]
  w2p = jnp.concatenate(
      [W2, jnp.zeros((D_H, DC2 - D_OUT), jnp.float32)], axis=1)
  m2 = _mm12_call(a1, hist, W1, b1.reshape(1, D_H), w2p)

  p = _agg_edges_call(src2d, dst2d, m2).reshape(2, N, DC2)
  return _final_call(p, hist, b2.reshape(1, D_OUT))


# speedup vs baseline: 1.0471x; 1.0471x over previous
import functools

import jax
import jax.numpy as jnp
from jax import lax
from jax.experimental import pallas as pl
from jax.experimental.pallas import tpu as pltpu
from jax.experimental.pallas import tpu_sc as plsc

N = 10000
E = 160000
D_IN = 256
D_H = 512
D_OUT = 40
DC = 128
DC2 = 128
NC = 2
NS = 16
RPT = N // NS
B = 128
EP = 1280
EB = EP // NS
HEB = EB // 2
N16 = N + 16

_mesh = functools.partial(
    plsc.VectorSubcoreMesh, core_axis_name="c", subcore_axis_name="s")


def _fill_const(buf, rows, cols, val):
  nz = cols // 16

  def body(r, _):
    for j in range(nz):
      buf[r, j * 16:(j + 1) * 16] = jnp.full((16,), val, jnp.float32)
    return 0

  lax.fori_loop(0, rows, body, 0)


def _zero_acc(zbuf, acc, r0):
  for j in range(25):
    pltpu.sync_copy(zbuf, acc.at[pl.ds(r0 + j * 25, 25)])


def _pipelined_pass(tab, acc, src_hbm, dst_hbm, e0, sv0, dv0, sv1, dv1,
                    m0, m1, g0, g1, s0, s1, nb):
  def loadidx(t, sv, dv):
    pltpu.sync_copy(src_hbm.at[pl.ds(e0 + t * B, B)], sv)
    pltpu.sync_copy(dst_hbm.at[pl.ds(e0 + t * B, B)], dv)

  def gfire(sv, m, g):
    pltpu.async_copy(tab.at[sv], m, g)

  def gwait(sv, m, g):
    pltpu.make_async_copy(tab.at[sv], m, g).wait()

  def sfire(dv, m, s):
    pltpu.async_copy(m, acc.at[dv], s, add=True)

  def swait(dv, m, s):
    pltpu.make_async_copy(m, acc.at[dv], s).wait()

  loadidx(0, sv0, dv0)
  gfire(sv0, m0, g0)
  loadidx(1, sv1, dv1)
  gfire(sv1, m1, g1)

  def body(i, _):
    t0 = 2 * i
    t1 = t0 + 1
    gwait(sv0, m0, g0)
    sfire(dv0, m0, s0)
    gwait(sv1, m1, g1)
    sfire(dv1, m1, s1)
    swait(dv0, m0, s0)

    @pl.when(t0 + 2 < nb)
    def _():
      loadidx(t0 + 2, sv0, dv0)
      gfire(sv0, m0, g0)

    swait(dv1, m1, s1)

    @pl.when(t1 + 2 < nb)
    def _():
      loadidx(t1 + 2, sv1, dv1)
      gfire(sv1, m1, g1)

    return 0

  lax.fori_loop(0, nb // 2, body, 0)


def _hist_call(ef_hbm_flat):
  @functools.partial(
      pl.kernel,
      out_type=jax.ShapeDtypeStruct((2, NS, RPT, DC), jnp.float32),
      mesh=_mesh(),
      scratch_types=[
          pltpu.VMEM((B,), jnp.int32),
          pltpu.VMEM((B,), jnp.int32),
          pltpu.VMEM((B, DC), jnp.float32),
          pltpu.VMEM((25, DC), jnp.float32),
          pltpu.VMEM_SHARED((N16, DC), jnp.float32),
          pltpu.SemaphoreType.DMA,
          pltpu.SemaphoreType.DMA,
      ],
  )
  def k(ef_hbm, out_hbm, dv0, dv1, onesb, zbuf, acc, s0, s1):
    cid = lax.axis_index("c")
    sid = lax.axis_index("s")
    _fill_const(onesb, B, DC, 1.0)
    _fill_const(zbuf, 25, DC, 0.0)
    r0 = sid * RPT
    _zero_acc(zbuf, acc, r0)
    e0 = cid * (EP * B) + sid * (EB * B)
    plsc.subcore_barrier()

    def loadidx(t, dv):
      pltpu.sync_copy(ef_hbm.at[pl.ds(e0 + t * B, B)], dv)

    def sfire(dv, sem):
      pltpu.async_copy(onesb, acc.at[dv], sem, add=True)

    def swait(dv, sem):
      pltpu.make_async_copy(onesb, acc.at[dv], sem).wait()

    loadidx(0, dv0)
    sfire(dv0, s0)
    loadidx(1, dv1)
    sfire(dv1, s1)

    def body(i, _):
      t0 = 2 * i
      t1 = t0 + 1
      swait(dv0, s0)

      @pl.when(t0 + 2 < EB)
      def _():
        loadidx(t0 + 2, dv0)
        sfire(dv0, s0)

      swait(dv1, s1)

      @pl.when(t1 + 2 < EB)
      def _():
        loadidx(t1 + 2, dv1)
        sfire(dv1, s1)

      return 0

    lax.fori_loop(0, EB // 2, body, 0)
    plsc.subcore_barrier()
    pltpu.sync_copy(acc.at[pl.ds(r0, RPT)], out_hbm.at[cid, sid])

  return k(ef_hbm_flat)


def _agg_cols_call(src1d, dst1d, tables):
  nt = len(tables)
  cpc = nt // 2

  @functools.partial(
      pl.kernel,
      out_type=[jax.ShapeDtypeStruct((NS, RPT, DC), jnp.float32)] * nt,
      mesh=_mesh(),
      scratch_types=[
          pltpu.VMEM((B,), jnp.int32),
          pltpu.VMEM((B,), jnp.int32),
          pltpu.VMEM((B,), jnp.int32),
          pltpu.VMEM((B,), jnp.int32),
          pltpu.VMEM((B, DC), jnp.float32),
          pltpu.VMEM((B, DC), jnp.float32),
          pltpu.VMEM((25, DC), jnp.float32),
          pltpu.VMEM_SHARED((N16, DC), jnp.float32),
          pltpu.SemaphoreType.DMA,
          pltpu.SemaphoreType.DMA,
          pltpu.SemaphoreType.DMA,
          pltpu.SemaphoreType.DMA,
      ],
  )
  def k(src_hbm, dst_hbm, *rest):
    tabs = rest[:nt]
    outs = rest[nt:2 * nt]
    (sv0, dv0, sv1, dv1, m0, m1, zbuf, acc, g0, g1, s0, s1) = rest[2 * nt:]
    cid = lax.axis_index("c")
    sid = lax.axis_index("s")
    _fill_const(zbuf, 25, DC, 0.0)
    r0 = sid * RPT
    e0 = sid * (EB * B)

    def copy_out(out):
      pltpu.sync_copy(acc.at[pl.ds(r0, RPT)], out.at[sid])

    def do_chunk(tab):
      _pipelined_pass(tab, acc, src_hbm, dst_hbm, e0, sv0, dv0, sv1, dv1,
                      m0, m1, g0, g1, s0, s1, EB)

    _zero_acc(zbuf, acc, r0)
    plsc.subcore_barrier()
    for i in range(cpc):
      @pl.when(cid == 0)
      def _():
        do_chunk(tabs[i])

      @pl.when(cid == 1)
      def _():
        do_chunk(tabs[cpc + i])

      plsc.subcore_barrier()

      @pl.when(cid == 0)
      def _():
        copy_out(outs[i])

      @pl.when(cid == 1)
      def _():
        copy_out(outs[cpc + i])

      if i < cpc - 1:
        _zero_acc(zbuf, acc, r0)
        plsc.subcore_barrier()

  return k(src1d, dst1d, *tables)


def _agg_edges_call(src1d, dst1d, table):
  eb = EP // (NC * NS)

  @functools.partial(
      pl.kernel,
      out_type=jax.ShapeDtypeStruct((2, NS, RPT, DC2), jnp.float32),
      mesh=_mesh(),
      scratch_types=[
          pltpu.VMEM((B,), jnp.int32),
          pltpu.VMEM((B,), jnp.int32),
          pltpu.VMEM((B,), jnp.int32),
          pltpu.VMEM((B,), jnp.int32),
          pltpu.VMEM((B, DC2), jnp.float32),
          pltpu.VMEM((B, DC2), jnp.float32),
          pltpu.VMEM((25, DC2), jnp.float32),
          pltpu.VMEM_SHARED((N16, DC2), jnp.float32),
          pltpu.SemaphoreType.DMA,
          pltpu.SemaphoreType.DMA,
          pltpu.SemaphoreType.DMA,
          pltpu.SemaphoreType.DMA,
      ],
  )
  def k(src_hbm, dst_hbm, tab, out_hbm, sv0, dv0, sv1, dv1, m0, m1, zbuf,
        acc, g0, g1, s0, s1):
    cid = lax.axis_index("c")
    sid = lax.axis_index("s")
    _fill_const(zbuf, 25, DC2, 0.0)
    r0 = sid * RPT
    e0 = (cid * NS + sid) * (eb * B)
    _zero_acc(zbuf, acc, r0)
    plsc.subcore_barrier()
    _pipelined_pass(tab, acc, src_hbm, dst_hbm, e0, sv0, dv0, sv1, dv1,
                    m0, m1, g0, g1, s0, s1, eb)
    plsc.subcore_barrier()
    pltpu.sync_copy(acc.at[pl.ds(r0, RPT)], out_hbm.at[cid, sid])

  return k(src1d, dst1d, table)


_R = 1000


def _norms(hist_blk):
  deg_o = hist_blk[0, :, 0:1]
  deg_i = hist_blk[1, :, 0:1]
  return (lax.rsqrt(jnp.maximum(deg_o, 1.0)),
          lax.rsqrt(jnp.maximum(deg_i, 1.0)))


def _ep0_call(x, hist):
  def body(x_ref, h_ref, o0_ref, o1_ref):
    no, _ = _norms(h_ref[...])
    xs = x_ref[...] * no
    o0_ref[...] = xs[:, :DC]
    o1_ref[...] = xs[:, DC:]

  return pl.pallas_call(
      body,
      grid=(N // _R,),
      in_specs=[
          pl.BlockSpec((_R, D_IN), lambda i: (i, 0)),
          pl.BlockSpec((2, _R, DC), lambda i: (0, i, 0)),
      ],
      out_specs=[pl.BlockSpec((_R, DC), lambda i: (i, 0))] * 2,
      out_shape=[jax.ShapeDtypeStruct((N16, DC), jnp.float32)] * 2,
  )(x, hist)


def _mm0_call(a0, a1, hist, w0, b0):
  def body(a0_ref, a1_ref, h_ref, w_ref, b_ref, *o_refs):
    no, ni = _norms(h_ref[...])
    a = jnp.concatenate([a0_ref[...], a1_ref[...]], axis=1) * ni
    h = jnp.dot(a, w_ref[...], preferred_element_type=jnp.float32,
                precision=lax.Precision.HIGHEST)
    h = jnp.maximum(h + b_ref[...], 0.0) * no
    for j in range(4):
      o_refs[j][...] = h[:, j * DC:(j + 1) * DC]

  return pl.pallas_call(
      body,
      grid=(N // _R,),
      in_specs=[
          pl.BlockSpec((_R, DC), lambda i: (i, 0)),
          pl.BlockSpec((_R, DC), lambda i: (i, 0)),
          pl.BlockSpec((2, _R, DC), lambda i: (0, i, 0)),
          pl.BlockSpec((D_IN, D_H), lambda i: (0, 0)),
          pl.BlockSpec((1, D_H), lambda i: (0, 0)),
      ],
      out_specs=[pl.BlockSpec((_R, DC), lambda i: (i, 0))] * 4,
      out_shape=[jax.ShapeDtypeStruct((N16, DC), jnp.float32)] * 4,
  )(a0, a1, hist, w0, b0)


def _mm12_call(aggs, hist, w1, b1, w2p):
  def body(a0_ref, a1_ref, a2_ref, a3_ref, h_ref, w1_ref, b1_ref, w2_ref,
           o_ref):
    no, ni = _norms(h_ref[...])
    a = jnp.concatenate(
        [a0_ref[...], a1_ref[...], a2_ref[...], a3_ref[...]], axis=1) * ni
    t = jnp.dot(a, w1_ref[...], preferred_element_type=jnp.float32,
                precision=lax.Precision.HIGHEST)
    t = jnp.maximum(t + b1_ref[...], 0.0) * no
    o_ref[...] = jnp.dot(t, w2_ref[...], preferred_element_type=jnp.float32,
                         precision=lax.Precision.HIGHEST)

  return pl.pallas_call(
      body,
      grid=(N // _R,),
      in_specs=[pl.BlockSpec((_R, DC), lambda i: (i, 0))] * 4 + [
          pl.BlockSpec((2, _R, DC), lambda i: (0, i, 0)),
          pl.BlockSpec((D_H, D_H), lambda i: (0, 0)),
          pl.BlockSpec((1, D_H), lambda i: (0, 0)),
          pl.BlockSpec((D_H, DC2), lambda i: (0, 0)),
      ],
      out_specs=pl.BlockSpec((_R, DC2), lambda i: (i, 0)),
      out_shape=jax.ShapeDtypeStruct((N16, DC2), jnp.float32),
  )(*aggs, hist, w1, b1, w2p)


def _final_call(p, hist, b2):
  def body(p_ref, h_ref, b_ref, o_ref):
    _, ni = _norms(h_ref[...])
    s = (p_ref[0] + p_ref[1])[:, :D_OUT]
    o_ref[...] = s * ni + b_ref[...]

  return pl.pallas_call(
      body,
      grid=(N // _R,),
      in_specs=[
          pl.BlockSpec((2, _R, DC2), lambda i: (0, i, 0)),
          pl.BlockSpec((2, _R, DC), lambda i: (0, i, 0)),
          pl.BlockSpec((1, D_OUT), lambda i: (0, 0)),
      ],
      out_specs=pl.BlockSpec((_R, D_OUT), lambda i: (i, 0)),
      out_shape=jax.ShapeDtypeStruct((N, D_OUT), jnp.float32),
  )(p, hist, b2)


def kernel(features, edge_index, W0, b0, W1, b1, W2, b2):
  ei = edge_index.astype(jnp.int32)
  pad = jnp.full((EP * B - E,), N, jnp.int32)
  src1d = jnp.concatenate([ei[0], pad])
  dst1d = jnp.concatenate([ei[1], pad])
  hist = _hist_call(jnp.concatenate([src1d, dst1d])).reshape(2, N, DC)

  xs0, xs1 = _ep0_call(features, hist)
  a00, a01 = _agg_cols_call(src1d, dst1d, [xs0, xs1])
  h1 = _mm0_call(a00.reshape(N, DC), a01.reshape(N, DC), hist, W0,
                 b0.reshape(1, D_H))

  a1 = _agg_cols_call(src1d, dst1d, list(h1))
  a1 = [a.reshape(N, DC) for a in a1]
  w2p = jnp.concatenate(
      [W2, jnp.zeros((D_H, DC2 - D_OUT), jnp.float32)], axis=1)
  m2 = _mm12_call(a1, hist, W1, b1.reshape(1, D_H), w2p)

  p = _agg_edges_call(src1d, dst1d, m2).reshape(2, N, DC2)
  return _final_call(p, hist, b2.reshape(1, D_OUT))


# speedup vs baseline: 2.1060x; 2.0112x over previous
import functools

import jax
import jax.numpy as jnp
from jax import lax
from jax.experimental import pallas as pl
from jax.experimental.pallas import tpu as pltpu
from jax.experimental.pallas import tpu_sc as plsc

N = 10000
E = 160000
D_IN = 256
D_H = 512
D_OUT = 40
DC = 128
DC2 = 128
NC = 2
NS = 16
RPT = N // NS
B = 128
EP = 1280
EB = EP // NS
HEB = EB // 2
N16 = N + 16

_mesh = functools.partial(
    plsc.VectorSubcoreMesh, core_axis_name="c", subcore_axis_name="s")


def _fill_const(buf, rows, cols, val):
  nz = cols // 16

  def body(r, _):
    for j in range(nz):
      buf[r, j * 16:(j + 1) * 16] = jnp.full((16,), val, jnp.float32)
    return 0

  lax.fori_loop(0, rows, body, 0)


def _zero_acc(zbuf, acc, r0):
  for j in range(25):
    pltpu.sync_copy(zbuf, acc.at[pl.ds(r0 + j * 25, 25)])


def _pipelined_pass(tab, acc, src_hbm, dst_hbm, e0, sv0, dv0, sv1, dv1,
                    m0, m1, g0, g1, s0, s1, nb):
  def loadidx(t, sv, dv):
    pltpu.sync_copy(src_hbm.at[pl.ds(e0 + t * B, B)], sv)
    pltpu.sync_copy(dst_hbm.at[pl.ds(e0 + t * B, B)], dv)

  def gfire(sv, m, g):
    pltpu.async_copy(tab.at[sv], m, g)

  def gwait(sv, m, g):
    pltpu.make_async_copy(tab.at[sv], m, g).wait()

  def sfire(dv, m, s):
    pltpu.async_copy(m, acc.at[dv], s, add=True)

  def swait(dv, m, s):
    pltpu.make_async_copy(m, acc.at[dv], s).wait()

  loadidx(0, sv0, dv0)
  gfire(sv0, m0, g0)
  loadidx(1, sv1, dv1)
  gfire(sv1, m1, g1)

  def body(i, _):
    t0 = 2 * i
    t1 = t0 + 1
    gwait(sv0, m0, g0)
    sfire(dv0, m0, s0)
    gwait(sv1, m1, g1)
    sfire(dv1, m1, s1)
    swait(dv0, m0, s0)

    @pl.when(t0 + 2 < nb)
    def _():
      loadidx(t0 + 2, sv0, dv0)
      gfire(sv0, m0, g0)

    swait(dv1, m1, s1)

    @pl.when(t1 + 2 < nb)
    def _():
      loadidx(t1 + 2, sv1, dv1)
      gfire(sv1, m1, g1)

    return 0

  lax.fori_loop(0, nb // 2, body, 0)


def _hist_call(ef_hbm_flat):
  @functools.partial(
      pl.kernel,
      out_type=jax.ShapeDtypeStruct((2, NS, RPT, DC), jnp.float32),
      mesh=_mesh(),
      scratch_types=[
          pltpu.VMEM((B,), jnp.int32),
          pltpu.VMEM((B,), jnp.int32),
          pltpu.VMEM((B, DC), jnp.float32),
          pltpu.VMEM((25, DC), jnp.float32),
          pltpu.VMEM_SHARED((N16, DC), jnp.float32),
          pltpu.SemaphoreType.DMA,
          pltpu.SemaphoreType.DMA,
      ],
  )
  def k(ef_hbm, out_hbm, dv0, dv1, onesb, zbuf, acc, s0, s1):
    cid = lax.axis_index("c")
    sid = lax.axis_index("s")
    _fill_const(onesb, B, DC, 1.0)
    _fill_const(zbuf, 25, DC, 0.0)
    r0 = sid * RPT
    _zero_acc(zbuf, acc, r0)
    e0 = cid * (EP * B) + sid * (EB * B)
    plsc.subcore_barrier()

    def loadidx(t, dv):
      pltpu.sync_copy(ef_hbm.at[pl.ds(e0 + t * B, B)], dv)

    def sfire(dv, sem):
      pltpu.async_copy(onesb, acc.at[dv], sem, add=True)

    def swait(dv, sem):
      pltpu.make_async_copy(onesb, acc.at[dv], sem).wait()

    loadidx(0, dv0)
    sfire(dv0, s0)
    loadidx(1, dv1)
    sfire(dv1, s1)

    def body(i, _):
      t0 = 2 * i
      t1 = t0 + 1
      swait(dv0, s0)

      @pl.when(t0 + 2 < EB)
      def _():
        loadidx(t0 + 2, dv0)
        sfire(dv0, s0)

      swait(dv1, s1)

      @pl.when(t1 + 2 < EB)
      def _():
        loadidx(t1 + 2, dv1)
        sfire(dv1, s1)

      return 0

    lax.fori_loop(0, EB // 2, body, 0)
    plsc.subcore_barrier()
    pltpu.sync_copy(acc.at[pl.ds(r0, RPT)], out_hbm.at[cid, sid])

  return k(ef_hbm_flat)


def _agg_cols_call(src1d, dst1d, tables):
  nt = len(tables)
  cpc = nt // 2

  @functools.partial(
      pl.kernel,
      out_type=[jax.ShapeDtypeStruct((NS, RPT, DC), jnp.float32)] * nt,
      mesh=_mesh(),
      scratch_types=[
          pltpu.VMEM((B,), jnp.int32),
          pltpu.VMEM((B,), jnp.int32),
          pltpu.VMEM((B,), jnp.int32),
          pltpu.VMEM((B,), jnp.int32),
          pltpu.VMEM((B, DC), jnp.float32),
          pltpu.VMEM((B, DC), jnp.float32),
          pltpu.VMEM((25, DC), jnp.float32),
          pltpu.VMEM_SHARED((N16, DC), jnp.float32),
          pltpu.SemaphoreType.DMA,
          pltpu.SemaphoreType.DMA,
          pltpu.SemaphoreType.DMA,
          pltpu.SemaphoreType.DMA,
      ],
  )
  def k(src_hbm, dst_hbm, *rest):
    tabs = rest[:nt]
    outs = rest[nt:2 * nt]
    (sv0, dv0, sv1, dv1, m0, m1, zbuf, acc, g0, g1, s0, s1) = rest[2 * nt:]
    cid = lax.axis_index("c")
    sid = lax.axis_index("s")
    _fill_const(zbuf, 25, DC, 0.0)
    r0 = sid * RPT
    e0 = sid * (EB * B)

    def copy_out(out):
      pltpu.sync_copy(acc.at[pl.ds(r0, RPT)], out.at[sid])

    def do_chunk(tab):
      _pipelined_pass(tab, acc, src_hbm, dst_hbm, e0, sv0, dv0, sv1, dv1,
                      m0, m1, g0, g1, s0, s1, EB)

    _zero_acc(zbuf, acc, r0)
    plsc.subcore_barrier()
    for i in range(cpc):
      @pl.when(cid == 0)
      def _():
        do_chunk(tabs[i])

      @pl.when(cid == 1)
      def _():
        do_chunk(tabs[cpc + i])

      plsc.subcore_barrier()

      @pl.when(cid == 0)
      def _():
        copy_out(outs[i])

      @pl.when(cid == 1)
      def _():
        copy_out(outs[cpc + i])

      if i < cpc - 1:
        _zero_acc(zbuf, acc, r0)
        plsc.subcore_barrier()

  return k(src1d, dst1d, *tables)


def _agg_edges_call(src1d, dst1d, table):
  eb = EP // (NC * NS)

  @functools.partial(
      pl.kernel,
      out_type=jax.ShapeDtypeStruct((2, NS, RPT, DC2), jnp.float32),
      mesh=_mesh(),
      scratch_types=[
          pltpu.VMEM((B,), jnp.int32),
          pltpu.VMEM((B,), jnp.int32),
          pltpu.VMEM((B,), jnp.int32),
          pltpu.VMEM((B,), jnp.int32),
          pltpu.VMEM((B, DC2), jnp.float32),
          pltpu.VMEM((B, DC2), jnp.float32),
          pltpu.VMEM((25, DC2), jnp.float32),
          pltpu.VMEM_SHARED((N16, DC2), jnp.float32),
          pltpu.SemaphoreType.DMA,
          pltpu.SemaphoreType.DMA,
          pltpu.SemaphoreType.DMA,
          pltpu.SemaphoreType.DMA,
      ],
  )
  def k(src_hbm, dst_hbm, tab, out_hbm, sv0, dv0, sv1, dv1, m0, m1, zbuf,
        acc, g0, g1, s0, s1):
    cid = lax.axis_index("c")
    sid = lax.axis_index("s")
    _fill_const(zbuf, 25, DC2, 0.0)
    r0 = sid * RPT
    e0 = (cid * NS + sid) * (eb * B)
    _zero_acc(zbuf, acc, r0)
    plsc.subcore_barrier()
    _pipelined_pass(tab, acc, src_hbm, dst_hbm, e0, sv0, dv0, sv1, dv1,
                    m0, m1, g0, g1, s0, s1, eb)
    plsc.subcore_barrier()
    pltpu.sync_copy(acc.at[pl.ds(r0, RPT)], out_hbm.at[cid, sid])

  return k(src1d, dst1d, table)


_R = 1000


def _norms(hist_blk):
  deg_o = hist_blk[0, :, 0:1]
  deg_i = hist_blk[1, :, 0:1]
  return (lax.rsqrt(jnp.maximum(deg_o, 1.0)),
          lax.rsqrt(jnp.maximum(deg_i, 1.0)))


def _ep0_call(x, hist):
  def body(x_ref, h_ref, o0_ref, o1_ref):
    no, _ = _norms(h_ref[...])
    xs = x_ref[...] * no
    o0_ref[...] = xs[:, :DC]
    o1_ref[...] = xs[:, DC:]

  return pl.pallas_call(
      body,
      grid=(N // _R,),
      in_specs=[
          pl.BlockSpec((_R, D_IN), lambda i: (i, 0)),
          pl.BlockSpec((2, _R, DC), lambda i: (0, i, 0)),
      ],
      out_specs=[pl.BlockSpec((_R, DC), lambda i: (i, 0))] * 2,
      out_shape=[jax.ShapeDtypeStruct((N16, DC), jnp.float32)] * 2,
  )(x, hist)


def _mm0_call(a0, a1, hist, w0, b0):
  def body(a0_ref, a1_ref, h_ref, w_ref, b_ref, *o_refs):
    no, ni = _norms(h_ref[...])
    a = jnp.concatenate([a0_ref[...], a1_ref[...]], axis=1) * ni
    h = jnp.dot(a, w_ref[...], preferred_element_type=jnp.float32,
                precision=lax.Precision.HIGHEST)
    h = jnp.maximum(h + b_ref[...], 0.0) * no
    for j in range(4):
      o_refs[j][...] = h[:, j * DC:(j + 1) * DC]

  return pl.pallas_call(
      body,
      grid=(N // _R,),
      in_specs=[
          pl.BlockSpec((_R, DC), lambda i: (i, 0)),
          pl.BlockSpec((_R, DC), lambda i: (i, 0)),
          pl.BlockSpec((2, _R, DC), lambda i: (0, i, 0)),
          pl.BlockSpec((D_IN, D_H), lambda i: (0, 0)),
          pl.BlockSpec((1, D_H), lambda i: (0, 0)),
      ],
      out_specs=[pl.BlockSpec((_R, DC), lambda i: (i, 0))] * 4,
      out_shape=[jax.ShapeDtypeStruct((N16, DC), jnp.float32)] * 4,
  )(a0, a1, hist, w0, b0)


def _mm12_call(aggs, hist, w1, b1, w2p):
  def body(a0_ref, a1_ref, a2_ref, a3_ref, h_ref, w1_ref, b1_ref, w2_ref,
           o_ref):
    no, ni = _norms(h_ref[...])
    a = jnp.concatenate(
        [a0_ref[...], a1_ref[...], a2_ref[...], a3_ref[...]], axis=1) * ni
    t = jnp.dot(a, w1_ref[...], preferred_element_type=jnp.float32,
                precision=lax.Precision.HIGHEST)
    t = jnp.maximum(t + b1_ref[...], 0.0) * no
    o_ref[...] = jnp.dot(t, w2_ref[...], preferred_element_type=jnp.float32,
                         precision=lax.Precision.HIGHEST)

  return pl.pallas_call(
      body,
      grid=(N // _R,),
      in_specs=[pl.BlockSpec((_R, DC), lambda i: (i, 0))] * 4 + [
          pl.BlockSpec((2, _R, DC), lambda i: (0, i, 0)),
          pl.BlockSpec((D_H, D_H), lambda i: (0, 0)),
          pl.BlockSpec((1, D_H), lambda i: (0, 0)),
          pl.BlockSpec((D_H, DC2), lambda i: (0, 0)),
      ],
      out_specs=pl.BlockSpec((_R, DC2), lambda i: (i, 0)),
      out_shape=jax.ShapeDtypeStruct((N16, DC2), jnp.float32),
  )(*aggs, hist, w1, b1, w2p)


def _final_call(p, hist, b2):
  def body(p_ref, h_ref, b_ref, o_ref):
    _, ni = _norms(h_ref[...])
    s = (p_ref[0] + p_ref[1])[:, :D_OUT]
    o_ref[...] = s * ni + b_ref[...]

  return pl.pallas_call(
      body,
      grid=(N // _R,),
      in_specs=[
          pl.BlockSpec((2, _R, DC2), lambda i: (0, i, 0)),
          pl.BlockSpec((2, _R, DC), lambda i: (0, i, 0)),
          pl.BlockSpec((1, D_OUT), lambda i: (0, 0)),
      ],
      out_specs=pl.BlockSpec((_R, D_OUT), lambda i: (i, 0)),
      out_shape=jax.ShapeDtypeStruct((N, D_OUT), jnp.float32),
  )(p, hist, b2)


def kernel(features, edge_index, W0, b0, W1, b1, W2, b2):
  ei = edge_index.astype(jnp.int32)
  pad = N + (jnp.arange(EP * B - E, dtype=jnp.int32) % 16)
  src1d = jnp.concatenate([ei[0], pad])
  dst1d = jnp.concatenate([ei[1], pad])
  hist = _hist_call(jnp.concatenate([src1d, dst1d])).reshape(2, N, DC)

  xs0, xs1 = _ep0_call(features, hist)
  a00, a01 = _agg_cols_call(src1d, dst1d, [xs0, xs1])
  h1 = _mm0_call(a00.reshape(N, DC), a01.reshape(N, DC), hist, W0,
                 b0.reshape(1, D_H))

  a1 = _agg_cols_call(src1d, dst1d, list(h1))
  a1 = [a.reshape(N, DC) for a in a1]
  w2p = jnp.concatenate(
      [W2, jnp.zeros((D_H, DC2 - D_OUT), jnp.float32)], axis=1)
  m2 = _mm12_call(a1, hist, W1, b1.reshape(1, D_H), w2p)

  p = _agg_edges_call(src1d, dst1d, m2).reshape(2, N, DC2)
  return _final_call(p, hist, b2.reshape(1, D_OUT))


# speedup vs baseline: 2.2732x; 1.0794x over previous
import functools

import jax
import jax.numpy as jnp
from jax import lax
from jax.experimental import pallas as pl
from jax.experimental.pallas import tpu as pltpu
from jax.experimental.pallas import tpu_sc as plsc

N = 10000
E = 160000
D_IN = 256
D_H = 512
D_OUT = 40
DC = 128
DC2 = 128
NC = 2
NS = 16
RPT = N // NS
B = 128
EP = 1280
EB = EP // NS
HEB = EB // 2
N16 = N + 16

_mesh = functools.partial(
    plsc.VectorSubcoreMesh, core_axis_name="c", subcore_axis_name="s")


def _fill_const(buf, rows, cols, val):
  nz = cols // 16

  def body(r, _):
    for j in range(nz):
      buf[r, j * 16:(j + 1) * 16] = jnp.full((16,), val, jnp.float32)
    return 0

  lax.fori_loop(0, rows, body, 0)


def _zero_acc(zbuf, acc, r0):
  for j in range(25):
    pltpu.sync_copy(zbuf, acc.at[pl.ds(r0 + j * 25, 25)])


def _pipelined_pass(tab, acc, src_hbm, dst_hbm, e0, sv0, dv0, sv1, dv1,
                    m0, m1, g0, g1, s0, s1, nb):
  def loadidx(t, sv, dv):
    pltpu.sync_copy(src_hbm.at[pl.ds(e0 + t * B, B)], sv)
    pltpu.sync_copy(dst_hbm.at[pl.ds(e0 + t * B, B)], dv)

  def gfire(sv, m, g):
    pltpu.async_copy(tab.at[sv], m, g)

  def gwait(sv, m, g):
    pltpu.make_async_copy(tab.at[sv], m, g).wait()

  def sfire(dv, m, s):
    pltpu.async_copy(m, acc.at[dv], s, add=True)

  def swait(dv, m, s):
    pltpu.make_async_copy(m, acc.at[dv], s).wait()

  loadidx(0, sv0, dv0)
  gfire(sv0, m0, g0)
  loadidx(1, sv1, dv1)
  gfire(sv1, m1, g1)

  def body(i, _):
    t0 = 2 * i
    t1 = t0 + 1
    gwait(sv0, m0, g0)
    sfire(dv0, m0, s0)
    gwait(sv1, m1, g1)
    sfire(dv1, m1, s1)
    swait(dv0, m0, s0)

    @pl.when(t0 + 2 < nb)
    def _():
      loadidx(t0 + 2, sv0, dv0)
      gfire(sv0, m0, g0)

    swait(dv1, m1, s1)

    @pl.when(t1 + 2 < nb)
    def _():
      loadidx(t1 + 2, sv1, dv1)
      gfire(sv1, m1, g1)

    return 0

  lax.fori_loop(0, nb // 2, body, 0)


def _hist_call(ef_hbm_flat):
  @functools.partial(
      pl.kernel,
      out_type=jax.ShapeDtypeStruct((2, NS, RPT, DC), jnp.float32),
      mesh=_mesh(),
      scratch_types=[
          pltpu.VMEM((B,), jnp.int32),
          pltpu.VMEM((B,), jnp.int32),
          pltpu.VMEM((B, DC), jnp.float32),
          pltpu.VMEM((25, DC), jnp.float32),
          pltpu.VMEM_SHARED((N16, DC), jnp.float32),
          pltpu.SemaphoreType.DMA,
          pltpu.SemaphoreType.DMA,
      ],
  )
  def k(ef_hbm, out_hbm, dv0, dv1, onesb, zbuf, acc, s0, s1):
    cid = lax.axis_index("c")
    sid = lax.axis_index("s")
    _fill_const(onesb, B, DC, 1.0)
    _fill_const(zbuf, 25, DC, 0.0)
    r0 = sid * RPT
    _zero_acc(zbuf, acc, r0)
    e0 = cid * (EP * B) + sid * (EB * B)
    plsc.subcore_barrier()

    def loadidx(t, dv):
      pltpu.sync_copy(ef_hbm.at[pl.ds(e0 + t * B, B)], dv)

    def sfire(dv, sem):
      pltpu.async_copy(onesb, acc.at[dv], sem, add=True)

    def swait(dv, sem):
      pltpu.make_async_copy(onesb, acc.at[dv], sem).wait()

    loadidx(0, dv0)
    sfire(dv0, s0)
    loadidx(1, dv1)
    sfire(dv1, s1)

    def body(i, _):
      t0 = 2 * i
      t1 = t0 + 1
      swait(dv0, s0)

      @pl.when(t0 + 2 < EB)
      def _():
        loadidx(t0 + 2, dv0)
        sfire(dv0, s0)

      swait(dv1, s1)

      @pl.when(t1 + 2 < EB)
      def _():
        loadidx(t1 + 2, dv1)
        sfire(dv1, s1)

      return 0

    lax.fori_loop(0, EB // 2, body, 0)
    plsc.subcore_barrier()
    pltpu.sync_copy(acc.at[pl.ds(r0, RPT)], out_hbm.at[cid, sid])

  return k(ef_hbm_flat)


def _agg_cols_call(src1d, dst1d, tables):
  nt = len(tables)
  cpc = nt // 2

  @functools.partial(
      pl.kernel,
      out_type=[jax.ShapeDtypeStruct((NS, RPT, DC), jnp.float32)] * nt,
      mesh=_mesh(),
      scratch_types=[
          pltpu.VMEM((B,), jnp.int32),
          pltpu.VMEM((B,), jnp.int32),
          pltpu.VMEM((B,), jnp.int32),
          pltpu.VMEM((B,), jnp.int32),
          pltpu.VMEM((B, DC), jnp.float32),
          pltpu.VMEM((B, DC), jnp.float32),
          pltpu.VMEM((25, DC), jnp.float32),
          pltpu.VMEM_SHARED((N16, DC), jnp.float32),
          pltpu.SemaphoreType.DMA,
          pltpu.SemaphoreType.DMA,
          pltpu.SemaphoreType.DMA,
          pltpu.SemaphoreType.DMA,
      ],
  )
  def k(src_hbm, dst_hbm, *rest):
    tabs = rest[:nt]
    outs = rest[nt:2 * nt]
    (sv0, dv0, sv1, dv1, m0, m1, zbuf, acc, g0, g1, s0, s1) = rest[2 * nt:]
    cid = lax.axis_index("c")
    sid = lax.axis_index("s")
    _fill_const(zbuf, 25, DC, 0.0)
    r0 = sid * RPT
    e0 = sid * (EB * B)

    def copy_out(out):
      pltpu.sync_copy(acc.at[pl.ds(r0, RPT)], out.at[sid])

    def do_chunk(tab):
      _pipelined_pass(tab, acc, src_hbm, dst_hbm, e0, sv0, dv0, sv1, dv1,
                      m0, m1, g0, g1, s0, s1, EB)

    _zero_acc(zbuf, acc, r0)
    plsc.subcore_barrier()
    for i in range(cpc):
      @pl.when(cid == 0)
      def _():
        do_chunk(tabs[i])

      @pl.when(cid == 1)
      def _():
        do_chunk(tabs[cpc + i])

      plsc.subcore_barrier()

      @pl.when(cid == 0)
      def _():
        copy_out(outs[i])

      @pl.when(cid == 1)
      def _():
        copy_out(outs[cpc + i])

      if i < cpc - 1:
        _zero_acc(zbuf, acc, r0)
        plsc.subcore_barrier()

  return k(src1d, dst1d, *tables)


def _agg_edges_call(src1d, dst1d, table):
  eb = EP // (NC * NS)

  @functools.partial(
      pl.kernel,
      out_type=jax.ShapeDtypeStruct((2, NS, RPT, DC2), jnp.float32),
      mesh=_mesh(),
      scratch_types=[
          pltpu.VMEM((B,), jnp.int32),
          pltpu.VMEM((B,), jnp.int32),
          pltpu.VMEM((B,), jnp.int32),
          pltpu.VMEM((B,), jnp.int32),
          pltpu.VMEM((B, DC2), jnp.float32),
          pltpu.VMEM((B, DC2), jnp.float32),
          pltpu.VMEM((25, DC2), jnp.float32),
          pltpu.VMEM_SHARED((N16, DC2), jnp.float32),
          pltpu.SemaphoreType.DMA,
          pltpu.SemaphoreType.DMA,
          pltpu.SemaphoreType.DMA,
          pltpu.SemaphoreType.DMA,
      ],
  )
  def k(src_hbm, dst_hbm, tab, out_hbm, sv0, dv0, sv1, dv1, m0, m1, zbuf,
        acc, g0, g1, s0, s1):
    cid = lax.axis_index("c")
    sid = lax.axis_index("s")
    _fill_const(zbuf, 25, DC2, 0.0)
    r0 = sid * RPT
    e0 = (cid * NS + sid) * (eb * B)
    _zero_acc(zbuf, acc, r0)
    plsc.subcore_barrier()
    _pipelined_pass(tab, acc, src_hbm, dst_hbm, e0, sv0, dv0, sv1, dv1,
                    m0, m1, g0, g1, s0, s1, eb)
    plsc.subcore_barrier()
    pltpu.sync_copy(acc.at[pl.ds(r0, RPT)], out_hbm.at[cid, sid])

  return k(src1d, dst1d, table)


_R = 1000


def _norms(hist_blk):
  deg_o = hist_blk[0, :, 0:1]
  deg_i = hist_blk[1, :, 0:1]
  return (lax.rsqrt(jnp.maximum(deg_o, 1.0)),
          lax.rsqrt(jnp.maximum(deg_i, 1.0)))


def _ep0_call(x, hist):
  def body(x_ref, h_ref, o0_ref, o1_ref):
    no, _ = _norms(h_ref[...])
    xs = x_ref[...] * no
    o0_ref[...] = xs[:, :DC]
    o1_ref[...] = xs[:, DC:]

  return pl.pallas_call(
      body,
      grid=(N // _R,),
      in_specs=[
          pl.BlockSpec((_R, D_IN), lambda i: (i, 0)),
          pl.BlockSpec((2, _R, DC), lambda i: (0, i, 0)),
      ],
      out_specs=[pl.BlockSpec((_R, DC), lambda i: (i, 0))] * 2,
      out_shape=[jax.ShapeDtypeStruct((N16, DC), jnp.float32)] * 2,
  )(x, hist)


def _mm0_call(a0, a1, hist, w0, b0):
  def body(a0_ref, a1_ref, h_ref, w_ref, b_ref, *o_refs):
    no, ni = _norms(h_ref[...])
    a = jnp.concatenate([a0_ref[...], a1_ref[...]], axis=1) * ni
    h = jnp.dot(a, w_ref[...], preferred_element_type=jnp.float32)
    h = jnp.maximum(h + b_ref[...], 0.0) * no
    for j in range(4):
      o_refs[j][...] = h[:, j * DC:(j + 1) * DC]

  return pl.pallas_call(
      body,
      grid=(N // _R,),
      in_specs=[
          pl.BlockSpec((_R, DC), lambda i: (i, 0)),
          pl.BlockSpec((_R, DC), lambda i: (i, 0)),
          pl.BlockSpec((2, _R, DC), lambda i: (0, i, 0)),
          pl.BlockSpec((D_IN, D_H), lambda i: (0, 0)),
          pl.BlockSpec((1, D_H), lambda i: (0, 0)),
      ],
      out_specs=[pl.BlockSpec((_R, DC), lambda i: (i, 0))] * 4,
      out_shape=[jax.ShapeDtypeStruct((N16, DC), jnp.float32)] * 4,
  )(a0, a1, hist, w0, b0)


def _mm12_call(aggs, hist, w1, b1, w2p):
  def body(a0_ref, a1_ref, a2_ref, a3_ref, h_ref, w1_ref, b1_ref, w2_ref,
           o_ref):
    no, ni = _norms(h_ref[...])
    a = jnp.concatenate(
        [a0_ref[...], a1_ref[...], a2_ref[...], a3_ref[...]], axis=1) * ni
    t = jnp.dot(a, w1_ref[...], preferred_element_type=jnp.float32)
    t = jnp.maximum(t + b1_ref[...], 0.0) * no
    o_ref[...] = jnp.dot(t, w2_ref[...], preferred_element_type=jnp.float32)

  return pl.pallas_call(
      body,
      grid=(N // _R,),
      in_specs=[pl.BlockSpec((_R, DC), lambda i: (i, 0))] * 4 + [
          pl.BlockSpec((2, _R, DC), lambda i: (0, i, 0)),
          pl.BlockSpec((D_H, D_H), lambda i: (0, 0)),
          pl.BlockSpec((1, D_H), lambda i: (0, 0)),
          pl.BlockSpec((D_H, DC2), lambda i: (0, 0)),
      ],
      out_specs=pl.BlockSpec((_R, DC2), lambda i: (i, 0)),
      out_shape=jax.ShapeDtypeStruct((N16, DC2), jnp.float32),
  )(*aggs, hist, w1, b1, w2p)


def _final_call(p, hist, b2):
  def body(p_ref, h_ref, b_ref, o_ref):
    _, ni = _norms(h_ref[...])
    s = (p_ref[0] + p_ref[1])[:, :D_OUT]
    o_ref[...] = s * ni + b_ref[...]

  return pl.pallas_call(
      body,
      grid=(N // _R,),
      in_specs=[
          pl.BlockSpec((2, _R, DC2), lambda i: (0, i, 0)),
          pl.BlockSpec((2, _R, DC), lambda i: (0, i, 0)),
          pl.BlockSpec((1, D_OUT), lambda i: (0, 0)),
      ],
      out_specs=pl.BlockSpec((_R, D_OUT), lambda i: (i, 0)),
      out_shape=jax.ShapeDtypeStruct((N, D_OUT), jnp.float32),
  )(p, hist, b2)


def kernel(features, edge_index, W0, b0, W1, b1, W2, b2):
  ei = edge_index.astype(jnp.int32)
  pad = N + (jnp.arange(EP * B - E, dtype=jnp.int32) % 16)
  src1d = jnp.concatenate([ei[0], pad])
  dst1d = jnp.concatenate([ei[1], pad])
  hist = _hist_call(jnp.concatenate([src1d, dst1d])).reshape(2, N, DC)

  xs0, xs1 = _ep0_call(features, hist)
  a00, a01 = _agg_cols_call(src1d, dst1d, [xs0, xs1])
  h1 = _mm0_call(a00.reshape(N, DC), a01.reshape(N, DC), hist, W0,
                 b0.reshape(1, D_H))

  a1 = _agg_cols_call(src1d, dst1d, list(h1))
  a1 = [a.reshape(N, DC) for a in a1]
  w2p = jnp.concatenate(
      [W2, jnp.zeros((D_H, DC2 - D_OUT), jnp.float32)], axis=1)
  m2 = _mm12_call(a1, hist, W1, b1.reshape(1, D_H), w2p)

  p = _agg_edges_call(src1d, dst1d, m2).reshape(2, N, DC2)
  return _final_call(p, hist, b2.reshape(1, D_OUT))


# speedup vs baseline: 2.3333x; 1.0264x over previous
import functools

import jax
import jax.numpy as jnp
from jax import lax
from jax.experimental import pallas as pl
from jax.experimental.pallas import tpu as pltpu
from jax.experimental.pallas import tpu_sc as plsc

N = 10000
E = 160000
D_IN = 256
D_H = 512
D_OUT = 40
DC = 128
DC2 = 128
NC = 2
NS = 16
RPT = N // NS
B = 128
EP = 1280
EB = EP // NS
HEB = EB // 2
N16 = N + 16

_mesh = functools.partial(
    plsc.VectorSubcoreMesh, core_axis_name="c", subcore_axis_name="s")


def _fill_const(buf, rows, cols, val):
  nz = cols // 16

  def body(r, _):
    for j in range(nz):
      buf[r, j * 16:(j + 1) * 16] = jnp.full((16,), val, jnp.float32)
    return 0

  lax.fori_loop(0, rows, body, 0)


def _zero_acc(zbuf, acc, r0):
  for j in range(25):
    pltpu.sync_copy(zbuf, acc.at[pl.ds(r0 + j * 25, 25)])


def _pipelined_pass(tab, acc, src_hbm, dst_hbm, e0, idx, m0, m1, g0, g1,
                    s0, s1, i0, i1, nb):
  def ifire(t, p, isem):
    sv, dv = idx[p]
    pltpu.async_copy(src_hbm.at[pl.ds(e0 + t * B, B)], sv, isem)
    pltpu.async_copy(dst_hbm.at[pl.ds(e0 + t * B, B)], dv, isem)

  def iwait(t, p, isem):
    sv, dv = idx[p]
    pltpu.make_async_copy(src_hbm.at[pl.ds(e0 + t * B, B)], sv, isem).wait()
    pltpu.make_async_copy(dst_hbm.at[pl.ds(e0 + t * B, B)], dv, isem).wait()

  def gfire(p, m, g):
    pltpu.async_copy(tab.at[idx[p][0]], m, g)

  def gwait(p, m, g):
    pltpu.make_async_copy(tab.at[idx[p][0]], m, g).wait()

  def sfire(p, m, sem):
    pltpu.async_copy(m, acc.at[idx[p][1]], sem, add=True)

  def swait(p, m, sem):
    pltpu.make_async_copy(m, acc.at[idx[p][1]], sem).wait()

  pltpu.sync_copy(src_hbm.at[pl.ds(e0, B)], idx[0][0])
  pltpu.sync_copy(dst_hbm.at[pl.ds(e0, B)], idx[0][1])
  pltpu.sync_copy(src_hbm.at[pl.ds(e0 + B, B)], idx[1][0])
  pltpu.sync_copy(dst_hbm.at[pl.ds(e0 + B, B)], idx[1][1])
  gfire(0, m0, g0)
  gfire(1, m1, g1)
  ifire(2, 2, i0)
  ifire(3, 3, i1)

  def body(i, _):
    t0 = 4 * i

    gwait(0, m0, g0)
    sfire(0, m0, s0)
    gwait(1, m1, g1)
    sfire(1, m1, s1)

    swait(0, m0, s0)
    iwait(t0 + 2, 2, i0)
    gfire(2, m0, g0)

    @pl.when(t0 + 4 < nb)
    def _():
      ifire(t0 + 4, 0, i0)

    swait(1, m1, s1)
    iwait(t0 + 3, 3, i1)
    gfire(3, m1, g1)

    @pl.when(t0 + 5 < nb)
    def _():
      ifire(t0 + 5, 1, i1)

    gwait(2, m0, g0)
    sfire(2, m0, s0)
    gwait(3, m1, g1)
    sfire(3, m1, s1)

    swait(2, m0, s0)

    @pl.when(t0 + 4 < nb)
    def _():
      iwait(t0 + 4, 0, i0)
      gfire(0, m0, g0)

    @pl.when(t0 + 6 < nb)
    def _():
      ifire(t0 + 6, 2, i0)

    swait(3, m1, s1)

    @pl.when(t0 + 5 < nb)
    def _():
      iwait(t0 + 5, 1, i1)
      gfire(1, m1, g1)

    @pl.when(t0 + 7 < nb)
    def _():
      ifire(t0 + 7, 3, i1)

    return 0

  lax.fori_loop(0, nb // 4, body, 0)


def _hist_call(ef_hbm_flat):
  @functools.partial(
      pl.kernel,
      out_type=jax.ShapeDtypeStruct((2, NS, RPT, DC), jnp.float32),
      mesh=_mesh(),
      scratch_types=[
          pltpu.VMEM((B,), jnp.int32),
          pltpu.VMEM((B,), jnp.int32),
          pltpu.VMEM((B, DC), jnp.float32),
          pltpu.VMEM((25, DC), jnp.float32),
          pltpu.VMEM_SHARED((N16, DC), jnp.float32),
          pltpu.SemaphoreType.DMA,
          pltpu.SemaphoreType.DMA,
      ],
  )
  def k(ef_hbm, out_hbm, dv0, dv1, onesb, zbuf, acc, s0, s1):
    cid = lax.axis_index("c")
    sid = lax.axis_index("s")
    _fill_const(onesb, B, DC, 1.0)
    _fill_const(zbuf, 25, DC, 0.0)
    r0 = sid * RPT
    _zero_acc(zbuf, acc, r0)
    e0 = cid * (EP * B) + sid * (EB * B)
    plsc.subcore_barrier()

    def loadidx(t, dv):
      pltpu.sync_copy(ef_hbm.at[pl.ds(e0 + t * B, B)], dv)

    def sfire(dv, sem):
      pltpu.async_copy(onesb, acc.at[dv], sem, add=True)

    def swait(dv, sem):
      pltpu.make_async_copy(onesb, acc.at[dv], sem).wait()

    loadidx(0, dv0)
    sfire(dv0, s0)
    loadidx(1, dv1)
    sfire(dv1, s1)

    def body(i, _):
      t0 = 2 * i
      t1 = t0 + 1
      swait(dv0, s0)

      @pl.when(t0 + 2 < EB)
      def _():
        loadidx(t0 + 2, dv0)
        sfire(dv0, s0)

      swait(dv1, s1)

      @pl.when(t1 + 2 < EB)
      def _():
        loadidx(t1 + 2, dv1)
        sfire(dv1, s1)

      return 0

    lax.fori_loop(0, EB // 2, body, 0)
    plsc.subcore_barrier()
    pltpu.sync_copy(acc.at[pl.ds(r0, RPT)], out_hbm.at[cid, sid])

  return k(ef_hbm_flat)


def _agg_cols_call(src1d, dst1d, tables):
  nt = len(tables)
  cpc = nt // 2

  @functools.partial(
      pl.kernel,
      out_type=[jax.ShapeDtypeStruct((NS, RPT, DC), jnp.float32)] * nt,
      mesh=_mesh(),
      scratch_types=[pltpu.VMEM((B,), jnp.int32)] * 8 + [
          pltpu.VMEM((B, DC), jnp.float32),
          pltpu.VMEM((B, DC), jnp.float32),
          pltpu.VMEM((25, DC), jnp.float32),
          pltpu.VMEM_SHARED((N16, DC), jnp.float32),
      ] + [pltpu.SemaphoreType.DMA] * 6,
  )
  def k(src_hbm, dst_hbm, *rest):
    tabs = rest[:nt]
    outs = rest[nt:2 * nt]
    (sv0, dv0, sv1, dv1, sv2, dv2, sv3, dv3, m0, m1, zbuf, acc,
     g0, g1, s0, s1, i0, i1) = rest[2 * nt:]
    idx = [(sv0, dv0), (sv1, dv1), (sv2, dv2), (sv3, dv3)]
    cid = lax.axis_index("c")
    sid = lax.axis_index("s")
    _fill_const(zbuf, 25, DC, 0.0)
    r0 = sid * RPT
    e0 = sid * (EB * B)

    def copy_out(out):
      pltpu.sync_copy(acc.at[pl.ds(r0, RPT)], out.at[sid])

    def do_chunk(tab):
      _pipelined_pass(tab, acc, src_hbm, dst_hbm, e0, idx, m0, m1,
                      g0, g1, s0, s1, i0, i1, EB)

    _zero_acc(zbuf, acc, r0)
    plsc.subcore_barrier()
    for i in range(cpc):
      @pl.when(cid == 0)
      def _():
        do_chunk(tabs[i])

      @pl.when(cid == 1)
      def _():
        do_chunk(tabs[cpc + i])

      plsc.subcore_barrier()

      @pl.when(cid == 0)
      def _():
        copy_out(outs[i])

      @pl.when(cid == 1)
      def _():
        copy_out(outs[cpc + i])

      if i < cpc - 1:
        _zero_acc(zbuf, acc, r0)
        plsc.subcore_barrier()

  return k(src1d, dst1d, *tables)


def _agg_edges_call(src1d, dst1d, table):
  eb = EP // (NC * NS)

  @functools.partial(
      pl.kernel,
      out_type=jax.ShapeDtypeStruct((2, NS, RPT, DC2), jnp.float32),
      mesh=_mesh(),
      scratch_types=[pltpu.VMEM((B,), jnp.int32)] * 8 + [
          pltpu.VMEM((B, DC2), jnp.float32),
          pltpu.VMEM((B, DC2), jnp.float32),
          pltpu.VMEM((25, DC2), jnp.float32),
          pltpu.VMEM_SHARED((N16, DC2), jnp.float32),
      ] + [pltpu.SemaphoreType.DMA] * 6,
  )
  def k(src_hbm, dst_hbm, tab, out_hbm, sv0, dv0, sv1, dv1, sv2, dv2, sv3,
        dv3, m0, m1, zbuf, acc, g0, g1, s0, s1, i0, i1):
    idx = [(sv0, dv0), (sv1, dv1), (sv2, dv2), (sv3, dv3)]
    cid = lax.axis_index("c")
    sid = lax.axis_index("s")
    _fill_const(zbuf, 25, DC2, 0.0)
    r0 = sid * RPT
    e0 = (cid * NS + sid) * (eb * B)
    _zero_acc(zbuf, acc, r0)
    plsc.subcore_barrier()
    _pipelined_pass(tab, acc, src_hbm, dst_hbm, e0, idx, m0, m1,
                    g0, g1, s0, s1, i0, i1, eb)
    plsc.subcore_barrier()
    pltpu.sync_copy(acc.at[pl.ds(r0, RPT)], out_hbm.at[cid, sid])

  return k(src1d, dst1d, table)


_R = 1000


def _norms(hist_blk):
  deg_o = hist_blk[0, :, 0:1]
  deg_i = hist_blk[1, :, 0:1]
  return (lax.rsqrt(jnp.maximum(deg_o, 1.0)),
          lax.rsqrt(jnp.maximum(deg_i, 1.0)))


def _ep0_call(x, hist):
  def body(x_ref, h_ref, o0_ref, o1_ref):
    no, _ = _norms(h_ref[...])
    xs = x_ref[...] * no
    o0_ref[...] = xs[:, :DC]
    o1_ref[...] = xs[:, DC:]

  return pl.pallas_call(
      body,
      grid=(N // _R,),
      in_specs=[
          pl.BlockSpec((_R, D_IN), lambda i: (i, 0)),
          pl.BlockSpec((2, _R, DC), lambda i: (0, i, 0)),
      ],
      out_specs=[pl.BlockSpec((_R, DC), lambda i: (i, 0))] * 2,
      out_shape=[jax.ShapeDtypeStruct((N16, DC), jnp.float32)] * 2,
  )(x, hist)


def _mm0_call(a0, a1, hist, w0, b0):
  def body(a0_ref, a1_ref, h_ref, w_ref, b_ref, *o_refs):
    no, ni = _norms(h_ref[...])
    a = jnp.concatenate([a0_ref[...], a1_ref[...]], axis=1) * ni
    h = jnp.dot(a, w_ref[...], preferred_element_type=jnp.float32)
    h = jnp.maximum(h + b_ref[...], 0.0) * no
    for j in range(4):
      o_refs[j][...] = h[:, j * DC:(j + 1) * DC]

  return pl.pallas_call(
      body,
      grid=(N // _R,),
      in_specs=[
          pl.BlockSpec((_R, DC), lambda i: (i, 0)),
          pl.BlockSpec((_R, DC), lambda i: (i, 0)),
          pl.BlockSpec((2, _R, DC), lambda i: (0, i, 0)),
          pl.BlockSpec((D_IN, D_H), lambda i: (0, 0)),
          pl.BlockSpec((1, D_H), lambda i: (0, 0)),
      ],
      out_specs=[pl.BlockSpec((_R, DC), lambda i: (i, 0))] * 4,
      out_shape=[jax.ShapeDtypeStruct((N16, DC), jnp.float32)] * 4,
  )(a0, a1, hist, w0, b0)


def _mm12_call(aggs, hist, w1, b1, w2p):
  def body(a0_ref, a1_ref, a2_ref, a3_ref, h_ref, w1_ref, b1_ref, w2_ref,
           o_ref):
    no, ni = _norms(h_ref[...])
    a = jnp.concatenate(
        [a0_ref[...], a1_ref[...], a2_ref[...], a3_ref[...]], axis=1) * ni
    t = jnp.dot(a, w1_ref[...], preferred_element_type=jnp.float32)
    t = jnp.maximum(t + b1_ref[...], 0.0) * no
    o_ref[...] = jnp.dot(t, w2_ref[...], preferred_element_type=jnp.float32)

  return pl.pallas_call(
      body,
      grid=(N // _R,),
      in_specs=[pl.BlockSpec((_R, DC), lambda i: (i, 0))] * 4 + [
          pl.BlockSpec((2, _R, DC), lambda i: (0, i, 0)),
          pl.BlockSpec((D_H, D_H), lambda i: (0, 0)),
          pl.BlockSpec((1, D_H), lambda i: (0, 0)),
          pl.BlockSpec((D_H, DC2), lambda i: (0, 0)),
      ],
      out_specs=pl.BlockSpec((_R, DC2), lambda i: (i, 0)),
      out_shape=jax.ShapeDtypeStruct((N16, DC2), jnp.float32),
  )(*aggs, hist, w1, b1, w2p)


def _final_call(p, hist, b2):
  def body(p_ref, h_ref, b_ref, o_ref):
    _, ni = _norms(h_ref[...])
    s = (p_ref[0] + p_ref[1])[:, :D_OUT]
    o_ref[...] = s * ni + b_ref[...]

  return pl.pallas_call(
      body,
      grid=(N // _R,),
      in_specs=[
          pl.BlockSpec((2, _R, DC2), lambda i: (0, i, 0)),
          pl.BlockSpec((2, _R, DC), lambda i: (0, i, 0)),
          pl.BlockSpec((1, D_OUT), lambda i: (0, 0)),
      ],
      out_specs=pl.BlockSpec((_R, D_OUT), lambda i: (i, 0)),
      out_shape=jax.ShapeDtypeStruct((N, D_OUT), jnp.float32),
  )(p, hist, b2)


def kernel(features, edge_index, W0, b0, W1, b1, W2, b2):
  ei = edge_index.astype(jnp.int32)
  pad = N + (jnp.arange(EP * B - E, dtype=jnp.int32) % 16)
  src1d = jnp.concatenate([ei[0], pad])
  dst1d = jnp.concatenate([ei[1], pad])
  hist = _hist_call(jnp.concatenate([src1d, dst1d])).reshape(2, N, DC)

  xs0, xs1 = _ep0_call(features, hist)
  a00, a01 = _agg_cols_call(src1d, dst1d, [xs0, xs1])
  h1 = _mm0_call(a00.reshape(N, DC), a01.reshape(N, DC), hist, W0,
                 b0.reshape(1, D_H))

  a1 = _agg_cols_call(src1d, dst1d, list(h1))
  a1 = [a.reshape(N, DC) for a in a1]
  w2p = jnp.concatenate(
      [W2, jnp.zeros((D_H, DC2 - D_OUT), jnp.float32)], axis=1)
  m2 = _mm12_call(a1, hist, W1, b1.reshape(1, D_H), w2p)

  p = _agg_edges_call(src1d, dst1d, m2).reshape(2, N, DC2)
  return _final_call(p, hist, b2.reshape(1, D_OUT))


# speedup vs baseline: 2.3546x; 1.0092x over previous
import functools

import jax
import jax.numpy as jnp
from jax import lax
from jax.experimental import pallas as pl
from jax.experimental.pallas import tpu as pltpu
from jax.experimental.pallas import tpu_sc as plsc

N = 10000
E = 160000
D_IN = 256
D_H = 512
D_OUT = 40
DC = 128
DC2 = 128
NC = 2
NS = 16
RPT = N // NS
B = 128
EP = 1280
EB = EP // NS
HEB = EB // 2
N16 = N + 16

_mesh = functools.partial(
    plsc.VectorSubcoreMesh, core_axis_name="c", subcore_axis_name="s")


def _fill_const(buf, rows, cols, val):
  nz = cols // 16

  def body(r, _):
    for j in range(nz):
      buf[r, j * 16:(j + 1) * 16] = jnp.full((16,), val, jnp.float32)
    return 0

  lax.fori_loop(0, rows, body, 0)


def _zero_acc(zbuf, acc, r0):
  for j in range(5):
    pltpu.sync_copy(zbuf, acc.at[pl.ds(r0 + j * 125, 125)])


def _pipelined_pass(tab, acc, src_hbm, dst_hbm, e0, idx, m0, m1, g0, g1,
                    s0, s1, i0, i1, nb):
  def ifire(t, p, isem):
    sv, dv = idx[p]
    pltpu.async_copy(src_hbm.at[pl.ds(e0 + t * B, B)], sv, isem)
    pltpu.async_copy(dst_hbm.at[pl.ds(e0 + t * B, B)], dv, isem)

  def iwait(t, p, isem):
    sv, dv = idx[p]
    pltpu.make_async_copy(src_hbm.at[pl.ds(e0 + t * B, B)], sv, isem).wait()
    pltpu.make_async_copy(dst_hbm.at[pl.ds(e0 + t * B, B)], dv, isem).wait()

  def gfire(p, m, g):
    pltpu.async_copy(tab.at[idx[p][0]], m, g)

  def gwait(p, m, g):
    pltpu.make_async_copy(tab.at[idx[p][0]], m, g).wait()

  def sfire(p, m, sem):
    pltpu.async_copy(m, acc.at[idx[p][1]], sem, add=True)

  def swait(p, m, sem):
    pltpu.make_async_copy(m, acc.at[idx[p][1]], sem).wait()

  pltpu.sync_copy(src_hbm.at[pl.ds(e0, B)], idx[0][0])
  pltpu.sync_copy(dst_hbm.at[pl.ds(e0, B)], idx[0][1])
  pltpu.sync_copy(src_hbm.at[pl.ds(e0 + B, B)], idx[1][0])
  pltpu.sync_copy(dst_hbm.at[pl.ds(e0 + B, B)], idx[1][1])
  gfire(0, m0, g0)
  gfire(1, m1, g1)
  ifire(2, 2, i0)
  ifire(3, 3, i1)

  def body(i, _):
    t0 = 4 * i

    gwait(0, m0, g0)
    sfire(0, m0, s0)
    gwait(1, m1, g1)
    sfire(1, m1, s1)

    swait(0, m0, s0)
    iwait(t0 + 2, 2, i0)
    gfire(2, m0, g0)

    @pl.when(t0 + 4 < nb)
    def _():
      ifire(t0 + 4, 0, i0)

    swait(1, m1, s1)
    iwait(t0 + 3, 3, i1)
    gfire(3, m1, g1)

    @pl.when(t0 + 5 < nb)
    def _():
      ifire(t0 + 5, 1, i1)

    gwait(2, m0, g0)
    sfire(2, m0, s0)
    gwait(3, m1, g1)
    sfire(3, m1, s1)

    swait(2, m0, s0)

    @pl.when(t0 + 4 < nb)
    def _():
      iwait(t0 + 4, 0, i0)
      gfire(0, m0, g0)

    @pl.when(t0 + 6 < nb)
    def _():
      ifire(t0 + 6, 2, i0)

    swait(3, m1, s1)

    @pl.when(t0 + 5 < nb)
    def _():
      iwait(t0 + 5, 1, i1)
      gfire(1, m1, g1)

    @pl.when(t0 + 7 < nb)
    def _():
      ifire(t0 + 7, 3, i1)

    return 0

  lax.fori_loop(0, nb // 4, body, 0)


def _hist_call(ef_hbm_flat):
  @functools.partial(
      pl.kernel,
      out_type=jax.ShapeDtypeStruct((2, NS, RPT, DC), jnp.float32),
      mesh=_mesh(),
      scratch_types=[
          pltpu.VMEM((B,), jnp.int32),
          pltpu.VMEM((B,), jnp.int32),
          pltpu.VMEM((B, DC), jnp.float32),
          pltpu.VMEM((125, DC), jnp.float32),
          pltpu.VMEM_SHARED((N16, DC), jnp.float32),
          pltpu.SemaphoreType.DMA,
          pltpu.SemaphoreType.DMA,
      ],
  )
  def k(ef_hbm, out_hbm, dv0, dv1, onesb, zbuf, acc, s0, s1):
    cid = lax.axis_index("c")
    sid = lax.axis_index("s")
    _fill_const(onesb, B, DC, 1.0)
    _fill_const(zbuf, 125, DC, 0.0)
    r0 = sid * RPT
    _zero_acc(zbuf, acc, r0)
    e0 = cid * (EP * B) + sid * (EB * B)
    plsc.subcore_barrier()

    def loadidx(t, dv):
      pltpu.sync_copy(ef_hbm.at[pl.ds(e0 + t * B, B)], dv)

    def sfire(dv, sem):
      pltpu.async_copy(onesb, acc.at[dv], sem, add=True)

    def swait(dv, sem):
      pltpu.make_async_copy(onesb, acc.at[dv], sem).wait()

    loadidx(0, dv0)
    sfire(dv0, s0)
    loadidx(1, dv1)
    sfire(dv1, s1)

    def body(i, _):
      t0 = 2 * i
      t1 = t0 + 1
      swait(dv0, s0)

      @pl.when(t0 + 2 < EB)
      def _():
        loadidx(t0 + 2, dv0)
        sfire(dv0, s0)

      swait(dv1, s1)

      @pl.when(t1 + 2 < EB)
      def _():
        loadidx(t1 + 2, dv1)
        sfire(dv1, s1)

      return 0

    lax.fori_loop(0, EB // 2, body, 0)
    plsc.subcore_barrier()
    pltpu.sync_copy(acc.at[pl.ds(r0, RPT)], out_hbm.at[cid, sid])

  return k(ef_hbm_flat)


def _agg_cols_call(src1d, dst1d, tables):
  nt = len(tables)
  cpc = nt // 2

  @functools.partial(
      pl.kernel,
      out_type=[jax.ShapeDtypeStruct((NS, RPT, DC), jnp.float32)] * nt,
      mesh=_mesh(),
      scratch_types=[pltpu.VMEM((B,), jnp.int32)] * 8 + [
          pltpu.VMEM((B, DC), jnp.float32),
          pltpu.VMEM((B, DC), jnp.float32),
          pltpu.VMEM((125, DC), jnp.float32),
          pltpu.VMEM_SHARED((N16, DC), jnp.float32),
      ] + [pltpu.SemaphoreType.DMA] * 6,
  )
  def k(src_hbm, dst_hbm, *rest):
    tabs = rest[:nt]
    outs = rest[nt:2 * nt]
    (sv0, dv0, sv1, dv1, sv2, dv2, sv3, dv3, m0, m1, zbuf, acc,
     g0, g1, s0, s1, i0, i1) = rest[2 * nt:]
    idx = [(sv0, dv0), (sv1, dv1), (sv2, dv2), (sv3, dv3)]
    cid = lax.axis_index("c")
    sid = lax.axis_index("s")
    _fill_const(zbuf, 125, DC, 0.0)
    r0 = sid * RPT
    e0 = sid * (EB * B)

    def copy_out(out):
      pltpu.sync_copy(acc.at[pl.ds(r0, RPT)], out.at[sid])

    def do_chunk(tab):
      _pipelined_pass(tab, acc, src_hbm, dst_hbm, e0, idx, m0, m1,
                      g0, g1, s0, s1, i0, i1, EB)

    _zero_acc(zbuf, acc, r0)
    plsc.subcore_barrier()
    for i in range(cpc):
      @pl.when(cid == 0)
      def _():
        do_chunk(tabs[i])

      @pl.when(cid == 1)
      def _():
        do_chunk(tabs[cpc + i])

      plsc.subcore_barrier()

      @pl.when(cid == 0)
      def _():
        copy_out(outs[i])

      @pl.when(cid == 1)
      def _():
        copy_out(outs[cpc + i])

      if i < cpc - 1:
        _zero_acc(zbuf, acc, r0)
        plsc.subcore_barrier()

  return k(src1d, dst1d, *tables)


def _agg_edges_call(src1d, dst1d, table):
  eb = EP // (NC * NS)

  @functools.partial(
      pl.kernel,
      out_type=jax.ShapeDtypeStruct((2, NS, RPT, DC2), jnp.float32),
      mesh=_mesh(),
      scratch_types=[pltpu.VMEM((B,), jnp.int32)] * 8 + [
          pltpu.VMEM((B, DC2), jnp.float32),
          pltpu.VMEM((B, DC2), jnp.float32),
          pltpu.VMEM((125, DC2), jnp.float32),
          pltpu.VMEM_SHARED((N16, DC2), jnp.float32),
      ] + [pltpu.SemaphoreType.DMA] * 6,
  )
  def k(src_hbm, dst_hbm, tab, out_hbm, sv0, dv0, sv1, dv1, sv2, dv2, sv3,
        dv3, m0, m1, zbuf, acc, g0, g1, s0, s1, i0, i1):
    idx = [(sv0, dv0), (sv1, dv1), (sv2, dv2), (sv3, dv3)]
    cid = lax.axis_index("c")
    sid = lax.axis_index("s")
    _fill_const(zbuf, 125, DC2, 0.0)
    r0 = sid * RPT
    e0 = (cid * NS + sid) * (eb * B)
    _zero_acc(zbuf, acc, r0)
    plsc.subcore_barrier()
    _pipelined_pass(tab, acc, src_hbm, dst_hbm, e0, idx, m0, m1,
                    g0, g1, s0, s1, i0, i1, eb)
    plsc.subcore_barrier()
    pltpu.sync_copy(acc.at[pl.ds(r0, RPT)], out_hbm.at[cid, sid])

  return k(src1d, dst1d, table)


_R = 1000


def _norms(hist_blk):
  deg_o = hist_blk[0, :, 0:1]
  deg_i = hist_blk[1, :, 0:1]
  return (lax.rsqrt(jnp.maximum(deg_o, 1.0)),
          lax.rsqrt(jnp.maximum(deg_i, 1.0)))


def _ep0_call(x, hist):
  def body(x_ref, h_ref, o0_ref, o1_ref):
    no, _ = _norms(h_ref[...])
    xs = x_ref[...] * no
    o0_ref[...] = xs[:, :DC]
    o1_ref[...] = xs[:, DC:]

  return pl.pallas_call(
      body,
      grid=(N // _R,),
      in_specs=[
          pl.BlockSpec((_R, D_IN), lambda i: (i, 0)),
          pl.BlockSpec((2, _R, DC), lambda i: (0, i, 0)),
      ],
      out_specs=[pl.BlockSpec((_R, DC), lambda i: (i, 0))] * 2,
      out_shape=[jax.ShapeDtypeStruct((N16, DC), jnp.float32)] * 2,
  )(x, hist)


def _mm0_call(a0, a1, hist, w0, b0):
  def body(a0_ref, a1_ref, h_ref, w_ref, b_ref, *o_refs):
    no, ni = _norms(h_ref[...])
    a = jnp.concatenate([a0_ref[...], a1_ref[...]], axis=1) * ni
    h = jnp.dot(a, w_ref[...], preferred_element_type=jnp.float32)
    h = jnp.maximum(h + b_ref[...], 0.0) * no
    for j in range(4):
      o_refs[j][...] = h[:, j * DC:(j + 1) * DC]

  return pl.pallas_call(
      body,
      grid=(N // _R,),
      in_specs=[
          pl.BlockSpec((_R, DC), lambda i: (i, 0)),
          pl.BlockSpec((_R, DC), lambda i: (i, 0)),
          pl.BlockSpec((2, _R, DC), lambda i: (0, i, 0)),
          pl.BlockSpec((D_IN, D_H), lambda i: (0, 0)),
          pl.BlockSpec((1, D_H), lambda i: (0, 0)),
      ],
      out_specs=[pl.BlockSpec((_R, DC), lambda i: (i, 0))] * 4,
      out_shape=[jax.ShapeDtypeStruct((N16, DC), jnp.float32)] * 4,
  )(a0, a1, hist, w0, b0)


def _mm12_call(aggs, hist, w1, b1, w2p):
  def body(a0_ref, a1_ref, a2_ref, a3_ref, h_ref, w1_ref, b1_ref, w2_ref,
           o_ref):
    no, ni = _norms(h_ref[...])
    a = jnp.concatenate(
        [a0_ref[...], a1_ref[...], a2_ref[...], a3_ref[...]], axis=1) * ni
    t = jnp.dot(a, w1_ref[...], preferred_element_type=jnp.float32)
    t = jnp.maximum(t + b1_ref[...], 0.0) * no
    o_ref[...] = jnp.dot(t, w2_ref[...], preferred_element_type=jnp.float32)

  return pl.pallas_call(
      body,
      grid=(N // _R,),
      in_specs=[pl.BlockSpec((_R, DC), lambda i: (i, 0))] * 4 + [
          pl.BlockSpec((2, _R, DC), lambda i: (0, i, 0)),
          pl.BlockSpec((D_H, D_H), lambda i: (0, 0)),
          pl.BlockSpec((1, D_H), lambda i: (0, 0)),
          pl.BlockSpec((D_H, DC2), lambda i: (0, 0)),
      ],
      out_specs=pl.BlockSpec((_R, DC2), lambda i: (i, 0)),
      out_shape=jax.ShapeDtypeStruct((N16, DC2), jnp.float32),
  )(*aggs, hist, w1, b1, w2p)


def _final_call(p, hist, b2):
  def body(p_ref, h_ref, b_ref, o_ref):
    _, ni = _norms(h_ref[...])
    s = (p_ref[0] + p_ref[1])[:, :D_OUT]
    o_ref[...] = s * ni + b_ref[...]

  return pl.pallas_call(
      body,
      grid=(N // _R,),
      in_specs=[
          pl.BlockSpec((2, _R, DC2), lambda i: (0, i, 0)),
          pl.BlockSpec((2, _R, DC), lambda i: (0, i, 0)),
          pl.BlockSpec((1, D_OUT), lambda i: (0, 0)),
      ],
      out_specs=pl.BlockSpec((_R, D_OUT), lambda i: (i, 0)),
      out_shape=jax.ShapeDtypeStruct((N, D_OUT), jnp.float32),
  )(p, hist, b2)


def kernel(features, edge_index, W0, b0, W1, b1, W2, b2):
  ei = edge_index.astype(jnp.int32)
  pad = N + (jnp.arange(EP * B - E, dtype=jnp.int32) % 16)
  src1d = jnp.concatenate([ei[0], pad])
  dst1d = jnp.concatenate([ei[1], pad])
  hist = _hist_call(jnp.concatenate([src1d, dst1d])).reshape(2, N, DC)

  xs0, xs1 = _ep0_call(features, hist)
  a00, a01 = _agg_cols_call(src1d, dst1d, [xs0, xs1])
  h1 = _mm0_call(a00.reshape(N, DC), a01.reshape(N, DC), hist, W0,
                 b0.reshape(1, D_H))

  a1 = _agg_cols_call(src1d, dst1d, list(h1))
  a1 = [a.reshape(N, DC) for a in a1]
  w2p = jnp.concatenate(
      [W2, jnp.zeros((D_H, DC2 - D_OUT), jnp.float32)], axis=1)
  m2 = _mm12_call(a1, hist, W1, b1.reshape(1, D_H), w2p)

  p = _agg_edges_call(src1d, dst1d, m2).reshape(2, N, DC2)
  return _final_call(p, hist, b2.reshape(1, D_OUT))
